# baseline (speedup 1.0000x reference)
import functools

import jax
import jax.numpy as jnp
from jax import lax
from jax.experimental import pallas as pl
from jax.experimental.pallas import tpu as pltpu
from jax.experimental.pallas import tpu_sc as plsc

_B = 32
_N = 100000
_L = 16
_CHUNK = 9984
_NMAIN = 10
_TAIL = _N - _NMAIN * _CHUNK
_UNROLL = 16
_BIG = 1e10


def _argmax_update(vals, idx, best_v, best_i):
    upd = vals > best_v
    return jnp.maximum(vals, best_v), jnp.where(upd, idx, best_i)


def _finalize_argmax(best_v, best_i):
    m = jnp.float32(-_BIG)
    im = jnp.int32(_N)
    for l in range(_L):
        v = best_v[l]
        ii = best_i[l]
        take = (v > m) | ((v == m) & (ii < im))
        m = jnp.where(take, v, m)
        im = jnp.where(take, ii, im)
    return im


def _lane(v, k):
    r = v[0]
    for l in range(1, _L):
        r = jnp.where(k == l, v[l], r)
    return r


def _fps_body(x_hbm, out_hbm, buf0, buf1, buf2, buft, wv, st, s0, s1, s2, st_sem):
    nc = 2
    b = lax.axis_index("s") * nc + lax.axis_index("c")
    lane = lax.iota(jnp.int32, _L)

    rbufs = (buf0, buf1, buf2)
    rsems = (s0, s1, s2)
    pass_slots = [
        (j * _CHUNK, _CHUNK, rbufs[j % 3], rsems[j % 3]) for j in range(_NMAIN)
    ] + [(_NMAIN * _CHUNK, _TAIL, buft, st_sem)]
    slots = pass_slots + pass_slots
    nslots = len(slots)
    boundary = len(pass_slots)

    def start(j):
        off, ln, dst, sem = slots[j]
        pltpu.async_copy(x_hbm.at[b, :, pl.ds(off, ln)], dst, sem)

    def wait(j):
        off, ln, dst, sem = slots[j]
        pltpu.make_async_copy(x_hbm.at[b, :, pl.ds(off, ln)], dst, sem).wait()

    def compute_a(off, ln, dst, carry):
        def body(i, c):
            vals = dst[1, pl.ds(i * _L, _L)]
            return _argmax_update(vals, lane + (off + i * _L), *c)

        return lax.fori_loop(0, ln // _L, body, carry, unroll=_UNROLL)

    def make_compute_b(c3):
        cx, cy, cz = c3

        def compute_b(off, ln, dst, carry):
            def body(i, c):
                vx = dst[0, pl.ds(i * _L, _L)]
                vy = dst[1, pl.ds(i * _L, _L)]
                vz = dst[2, pl.ds(i * _L, _L)]
                dx = vx - cx
                dy = vy - cy
                dz = vz - cz
                d = dx * dx + dy * dy + dz * dz
                return _argmax_update(d, lane + (off + i * _L), *c)

            return lax.fori_loop(0, ln // _L, body, carry, unroll=_UNROLL)

        return compute_b

    zero_carry = (
        jnp.full((_L,), -_BIG, jnp.float32),
        jnp.zeros((_L,), jnp.int32),
    )

    start(0)
    start(1)
    carry = zero_carry
    i0 = None
    compute = compute_a
    for j in range(nslots):
        if j == boundary:
            i0 = _finalize_argmax(*carry)
            wbase = pl.multiple_of((i0 // 128) * 128, 128)
            pltpu.sync_copy(x_hbm.at[b, :, pl.ds(wbase, 128)], wv)
            woff = i0 - wbase
            w8 = pl.multiple_of(jnp.minimum((woff // 8) * 8, 128 - _L), 8)
            wk = woff - w8
            c3 = tuple(
                jnp.full((_L,), _lane(wv[r, pl.ds(w8, _L)], wk), jnp.float32)
                for r in range(3)
            )
            compute = make_compute_b(c3)
            carry = zero_carry
        off, ln, dst, _ = slots[j]
        wait(j)
        carry = compute(off, ln, dst, carry)
        if j + 2 < nslots:
            start(j + 2)
    i1 = _finalize_argmax(*carry)

    res = jnp.where(lane == 0, i0, jnp.where(lane == 1, i1, 0))
    st[...] = res.reshape(1, _L)
    pltpu.sync_copy(st, out_hbm.at[b])


def kernel(xyz):
    x = xyz.reshape(_B, 3, _N)
    mesh = plsc.VectorSubcoreMesh(core_axis_name="c", subcore_axis_name="s")
    fps = functools.partial(
        pl.kernel,
        mesh=mesh,
        out_type=jax.ShapeDtypeStruct((_B, 1, _L), jnp.int32),
        scratch_types=[
            pltpu.VMEM((3, _CHUNK), jnp.float32),
            pltpu.VMEM((3, _CHUNK), jnp.float32),
            pltpu.VMEM((3, _CHUNK), jnp.float32),
            pltpu.VMEM((3, _TAIL), jnp.float32),
            pltpu.VMEM((3, 128), jnp.float32),
            pltpu.VMEM((1, _L), jnp.int32),
            pltpu.SemaphoreType.DMA,
            pltpu.SemaphoreType.DMA,
            pltpu.SemaphoreType.DMA,
            pltpu.SemaphoreType.DMA,
        ],
    )(_fps_body)
    out = fps(x)
    return out[:, 0, :2]


# speedup vs baseline: 1.0794x; 1.0794x over previous
import jax
import jax.numpy as jnp
from jax.experimental import pallas as pl
from jax.experimental.pallas import tpu as pltpu

_B = 32
_N = 100000
_G = 8


def _fps_kernel(x_ref, out_ref):
    xr = x_ref[:, 0, :]
    yr = x_ref[:, 1, :]
    zr = x_ref[:, 2, :]
    iota = jax.lax.broadcasted_iota(jnp.int32, (_G, _N), 1)

    m0 = jnp.max(yr, axis=1, keepdims=True)
    i0 = jnp.min(jnp.where(yr == m0, iota, _N), axis=1, keepdims=True)

    selc = iota == i0
    cx = jnp.sum(jnp.where(selc, xr, 0.0), axis=1, keepdims=True)
    cz = jnp.sum(jnp.where(selc, zr, 0.0), axis=1, keepdims=True)
    cy = m0

    dx = xr - cx
    dy = yr - cy
    dz = zr - cz
    d = jnp.minimum(dx * dx + dy * dy + dz * dz, 1e10)
    m1 = jnp.max(d, axis=1, keepdims=True)
    i1 = jnp.min(jnp.where(d == m1, iota, _N), axis=1, keepdims=True)

    out_ref[0] = jnp.concatenate([i0, i1], axis=1)


def kernel(xyz):
    ng = _B // _G
    x = xyz.reshape(_B, 3, _N)
    out = pl.pallas_call(
        _fps_kernel,
        grid=(ng,),
        in_specs=[pl.BlockSpec((_G, 3, _N), lambda g: (g, 0, 0))],
        out_specs=pl.BlockSpec((1, _G, 2), lambda g: (g, 0, 0)),
        out_shape=jax.ShapeDtypeStruct((ng, _G, 2), jnp.int32),
        compiler_params=pltpu.CompilerParams(
            dimension_semantics=("arbitrary",),
        ),
    )(x)
    return out.reshape(_B, 2)


# speedup vs baseline: 1.4509x; 1.3442x over previous
import jax
import jax.numpy as jnp
from jax.experimental import pallas as pl
from jax.experimental.pallas import tpu as pltpu

_B = 32
_N = 100000
_G = 8


def _fps_kernel(x_ref, out_ref):
    xr = x_ref[0, :, 0, :]
    yr = x_ref[0, :, 1, :]
    zr = x_ref[0, :, 2, :]
    iota = jax.lax.broadcasted_iota(jnp.int32, (_G, _N), 1)

    m0 = jnp.max(yr, axis=1, keepdims=True)
    i0 = jnp.min(jnp.where(yr == m0, iota, _N), axis=1, keepdims=True)

    selc = iota == i0
    cx = jnp.sum(jnp.where(selc, xr, 0.0), axis=1, keepdims=True)
    cz = jnp.sum(jnp.where(selc, zr, 0.0), axis=1, keepdims=True)
    cy = m0

    dx = xr - cx
    dy = yr - cy
    dz = zr - cz
    d = jnp.minimum(dx * dx + dy * dy + dz * dz, 1e10)
    m1 = jnp.max(d, axis=1, keepdims=True)
    i1 = jnp.min(jnp.where(d == m1, iota, _N), axis=1, keepdims=True)

    out_ref[0] = jnp.concatenate([i0, i1], axis=1)


def kernel(xyz):
    ng = _B // _G
    out = pl.pallas_call(
        _fps_kernel,
        grid=(ng,),
        in_specs=[pl.BlockSpec((1, _G, 3, _N), lambda g: (0, g, 0, 0))],
        out_specs=pl.BlockSpec((1, _G, 2), lambda g: (g, 0, 0)),
        out_shape=jax.ShapeDtypeStruct((ng, _G, 2), jnp.int32),
        compiler_params=pltpu.CompilerParams(
            dimension_semantics=("arbitrary",),
        ),
    )(xyz)
    return out.reshape(_B, 2)
